# Initial kernel scaffold; baseline (speedup 1.0000x reference)
import functools

import jax
import jax.numpy as jnp
from jax import lax
from jax.experimental import pallas as pl
from jax.experimental.pallas import tpu as pltpu
from jax.experimental.pallas import tpu_sc as plsc

N = 10000
D = 128
E = 320000
EPS = 1e-5

CHUNK = 128
NC, NS = 2, 16
NW = NC * NS
NUM_CHUNKS = 2560
E_PAD = NUM_CHUNKS * CHUNK
CHUNKS_PER_TILE = NUM_CHUNKS // NW
ZROWS = 64
ROWS_PER_TILE_SP = 626
N_SP = NS * ROWS_PER_TILE_SP
OUT_ROWS_PER_TILE = N // NS


def _linrelu_body(x_ref, wt_ref, b_ref, o_ref):
    acc = jnp.dot(x_ref[...], wt_ref[...], preferred_element_type=jnp.float32)
    o_ref[...] = jnp.maximum(acc + b_ref[...], 0.0)


def _linrelu(x, wt, b2):
    return pl.pallas_call(
        _linrelu_body,
        grid=(10,),
        in_specs=[
            pl.BlockSpec((N // 10, D), lambda i: (i, 0)),
            pl.BlockSpec((D, D), lambda i: (0, 0)),
            pl.BlockSpec((1, D), lambda i: (0, 0)),
        ],
        out_specs=pl.BlockSpec((N // 10, D), lambda i: (i, 0)),
        out_shape=jax.ShapeDtypeStruct((N, D), jnp.float32),
    )(x, wt, b2)


def _norm_body(x_ref, a0_ref, a1_ref, g_ref, bt_ref, o_ref):
    h = x_ref[...] + a0_ref[...] + a1_ref[...]
    ms = jnp.mean(h * h, axis=-1, keepdims=True)
    o_ref[...] = h * lax.rsqrt(ms + EPS) * g_ref[...] + bt_ref[...]


def _norm(x, a0, a1, g2, beta2):
    return pl.pallas_call(
        _norm_body,
        grid=(10,),
        in_specs=[
            pl.BlockSpec((N // 10, D), lambda i: (i, 0)),
            pl.BlockSpec((N // 10, D), lambda i: (i, 0)),
            pl.BlockSpec((N // 10, D), lambda i: (i, 0)),
            pl.BlockSpec((1, D), lambda i: (0, 0)),
            pl.BlockSpec((1, D), lambda i: (0, 0)),
        ],
        out_specs=pl.BlockSpec((N // 10, D), lambda i: (i, 0)),
        out_shape=jax.ShapeDtypeStruct((N, D), jnp.float32),
    )(x, a0, a1, g2, beta2)


@functools.partial(
    pl.kernel,
    out_type=jax.ShapeDtypeStruct((NC, N, D), jnp.float32),
    mesh=plsc.VectorSubcoreMesh(core_axis_name="c", subcore_axis_name="s"),
    scratch_types=[
        pltpu.VMEM((2, CHUNK), jnp.int32),
        pltpu.VMEM((2, CHUNK), jnp.int32),
        pltpu.VMEM((2, CHUNK, D), jnp.float32),
        pltpu.VMEM((ZROWS, D), jnp.float32),
        pltpu.VMEM_SHARED((N_SP, D), jnp.float32),
        pltpu.SemaphoreType.DMA,
    ],
)
def _sc_agg(m_hbm, src_hbm, dst_hbm, out_hbm, idx_s, idx_d, rows, zbuf, shared, sem):
    c = lax.axis_index("c")
    s = lax.axis_index("s")
    wid = c * NS + s

    @pl.loop(0, ZROWS)
    def _zrow(r):
        @pl.loop(0, D, step=16)
        def _zcol(col):
            zbuf[r, pl.ds(col, 16)] = jnp.zeros((16,), jnp.float32)

    zbase = s * ROWS_PER_TILE_SP

    @pl.loop(0, (ROWS_PER_TILE_SP // ZROWS) * ZROWS, step=ZROWS)
    def _zspm(r0):
        pltpu.sync_copy(zbuf, shared.at[pl.ds(zbase + r0, ZROWS)])

    _rem = ROWS_PER_TILE_SP % ZROWS
    pltpu.sync_copy(
        zbuf.at[pl.ds(0, _rem)],
        shared.at[pl.ds(zbase + (ROWS_PER_TILE_SP // ZROWS) * ZROWS, _rem)],
    )

    plsc.subcore_barrier()

    @pl.loop(0, CHUNKS_PER_TILE)
    def _edge(i):
        q = wid * CHUNKS_PER_TILE + i
        pltpu.sync_copy(src_hbm.at[q], idx_s.at[0])
        pltpu.sync_copy(dst_hbm.at[q], idx_d.at[0])
        pltpu.async_copy(m_hbm.at[idx_s.at[0]], rows.at[0], sem).wait()
        pltpu.sync_copy(rows.at[0], shared.at[idx_d.at[0]], add=True)

    plsc.subcore_barrier()

    obase = s * OUT_ROWS_PER_TILE
    pltpu.sync_copy(
        shared.at[pl.ds(obase, OUT_ROWS_PER_TILE)],
        out_hbm.at[c, pl.ds(obase, OUT_ROWS_PER_TILE)],
    )


def kernel(x, edge_index, W, b, g, beta):
    wt = W.T
    b2 = b.reshape(1, D)
    g2 = g.reshape(1, D)
    beta2 = beta.reshape(1, D)
    m = _linrelu(x, wt, b2)
    pad = E_PAD - E
    src = jnp.concatenate(
        [edge_index[0], jnp.zeros((pad,), jnp.int32)]).reshape(NUM_CHUNKS, CHUNK)
    dst = jnp.concatenate(
        [edge_index[1], jnp.full((pad,), N, jnp.int32)]).reshape(NUM_CHUNKS, CHUNK)
    parts = _sc_agg(m, src, dst)
    return _norm(x, parts[0], parts[1], g2, beta2)


# speedup vs baseline: 2.6575x; 2.6575x over previous
import functools

import jax
import jax.numpy as jnp
from jax import lax
from jax.experimental import pallas as pl
from jax.experimental.pallas import tpu as pltpu
from jax.experimental.pallas import tpu_sc as plsc

N = 10000
D = 128
E = 320000
EPS = 1e-5

CHUNK = 128
NC, NS = 2, 16
NW = NC * NS
NUM_CHUNKS = 2560
E_PAD = NUM_CHUNKS * CHUNK
CHUNKS_PER_TILE = NUM_CHUNKS // NW
ZROWS = 64
ROWS_PER_TILE_SP = 632
N_SP = NS * ROWS_PER_TILE_SP


def _linrelu_body(x_ref, wt_ref, b_ref, o_ref):
    acc = jnp.dot(x_ref[...], wt_ref[...], preferred_element_type=jnp.float32)
    o_ref[...] = jnp.maximum(acc + b_ref[...], 0.0)


def _linrelu(x, wt, b2):
    return pl.pallas_call(
        _linrelu_body,
        grid=(10,),
        in_specs=[
            pl.BlockSpec((N // 10, D), lambda i: (i, 0)),
            pl.BlockSpec((D, D), lambda i: (0, 0)),
            pl.BlockSpec((1, D), lambda i: (0, 0)),
        ],
        out_specs=pl.BlockSpec((N // 10, D), lambda i: (i, 0)),
        out_shape=jax.ShapeDtypeStruct((N, D), jnp.float32),
    )(x, wt, b2)


def _norm_body(x_ref, a0_ref, a1_ref, g_ref, bt_ref, o_ref):
    h = x_ref[...] + a0_ref[...] + a1_ref[...]
    ms = jnp.mean(h * h, axis=-1, keepdims=True)
    o_ref[...] = h * lax.rsqrt(ms + EPS) * g_ref[...] + bt_ref[...]


def _norm(x, a0, a1, g2, beta2):
    return pl.pallas_call(
        _norm_body,
        grid=(10,),
        in_specs=[
            pl.BlockSpec((N // 10, D), lambda i: (i, 0)),
            pl.BlockSpec((N // 10, D), lambda i: (i, 0)),
            pl.BlockSpec((N // 10, D), lambda i: (i, 0)),
            pl.BlockSpec((1, D), lambda i: (0, 0)),
            pl.BlockSpec((1, D), lambda i: (0, 0)),
        ],
        out_specs=pl.BlockSpec((N // 10, D), lambda i: (i, 0)),
        out_shape=jax.ShapeDtypeStruct((N, D), jnp.float32),
    )(x, a0, a1, g2, beta2)


@functools.partial(
    pl.kernel,
    out_type=jax.ShapeDtypeStruct((NC, N_SP, D), jnp.float32),
    mesh=plsc.VectorSubcoreMesh(core_axis_name="c", subcore_axis_name="s"),
    scratch_types=[
        pltpu.VMEM((2, CHUNK), jnp.int32),
        pltpu.VMEM((2, CHUNK), jnp.int32),
        pltpu.VMEM((2, CHUNK, D), jnp.float32),
        pltpu.VMEM((ZROWS, D), jnp.float32),
        pltpu.VMEM_SHARED((N_SP, D), jnp.float32),
        pltpu.SemaphoreType.DMA,
    ],
)
def _sc_agg(m_hbm, src_hbm, dst_hbm, out_hbm, idx_s, idx_d, rows, zbuf, shared, sem):
    c = lax.axis_index("c")
    s = lax.axis_index("s")
    wid = c * NS + s

    @pl.loop(0, ZROWS)
    def _zrow(r):
        @pl.loop(0, D, step=16)
        def _zcol(col):
            zbuf[r, pl.ds(col, 16)] = jnp.zeros((16,), jnp.float32)

    zbase = s * ROWS_PER_TILE_SP

    @pl.loop(0, (ROWS_PER_TILE_SP // ZROWS) * ZROWS, step=ZROWS)
    def _zspm(r0):
        pltpu.sync_copy(zbuf, shared.at[pl.ds(zbase + r0, ZROWS)])

    _rem = ROWS_PER_TILE_SP % ZROWS
    pltpu.sync_copy(
        zbuf.at[pl.ds(0, _rem)],
        shared.at[pl.ds(zbase + (ROWS_PER_TILE_SP // ZROWS) * ZROWS, _rem)],
    )

    plsc.subcore_barrier()

    @pl.loop(0, CHUNKS_PER_TILE)
    def _edge(i):
        e0 = (wid * CHUNKS_PER_TILE + i) * CHUNK
        pltpu.sync_copy(src_hbm.at[pl.ds(e0, CHUNK)], idx_s.at[0])
        pltpu.sync_copy(dst_hbm.at[pl.ds(e0, CHUNK)], idx_d.at[0])
        pltpu.async_copy(m_hbm.at[idx_s.at[0]], rows.at[0], sem).wait()
        pltpu.sync_copy(rows.at[0], shared.at[idx_d.at[0]], add=True)

    plsc.subcore_barrier()

    obase = s * ROWS_PER_TILE_SP
    pltpu.sync_copy(
        shared.at[pl.ds(obase, ROWS_PER_TILE_SP)],
        out_hbm.at[c, pl.ds(obase, ROWS_PER_TILE_SP)],
    )


def kernel(x, edge_index, W, b, g, beta):
    wt = W.T
    b2 = b.reshape(1, D)
    g2 = g.reshape(1, D)
    beta2 = beta.reshape(1, D)
    m = _linrelu(x, wt, b2)
    pad = E_PAD - E
    src = jnp.concatenate([edge_index[0], jnp.zeros((pad,), jnp.int32)])
    dst = jnp.concatenate([edge_index[1], jnp.full((pad,), N, jnp.int32)])
    parts = _sc_agg(m, src, dst)
    return _norm(x, parts[0, :N], parts[1, :N], g2, beta2)


# speedup vs baseline: 3.4278x; 1.2898x over previous
import functools

import jax
import jax.numpy as jnp
from jax import lax
from jax.experimental import pallas as pl
from jax.experimental.pallas import tpu as pltpu
from jax.experimental.pallas import tpu_sc as plsc

N = 10000
D = 128
E = 320000
EPS = 1e-5

CHUNK = 128
NC, NS = 2, 16
NW = NC * NS
NUM_CHUNKS = 2560
E_PAD = NUM_CHUNKS * CHUNK
CHUNKS_PER_TILE = NUM_CHUNKS // NW
ZROWS = 64
ROWS_PER_TILE_SP = 632
N_SP = NS * ROWS_PER_TILE_SP


def _linrelu_body(x_ref, wt_ref, b_ref, o_ref):
    acc = jnp.dot(x_ref[...], wt_ref[...], preferred_element_type=jnp.float32)
    o_ref[...] = jnp.maximum(acc + b_ref[...], 0.0)


def _linrelu(x, wt, b2):
    return pl.pallas_call(
        _linrelu_body,
        grid=(10,),
        in_specs=[
            pl.BlockSpec((N // 10, D), lambda i: (i, 0)),
            pl.BlockSpec((D, D), lambda i: (0, 0)),
            pl.BlockSpec((1, D), lambda i: (0, 0)),
        ],
        out_specs=pl.BlockSpec((N // 10, D), lambda i: (i, 0)),
        out_shape=jax.ShapeDtypeStruct((N, D), jnp.float32),
    )(x, wt, b2)


def _norm_body(x_ref, a0_ref, a1_ref, g_ref, bt_ref, o_ref):
    h = x_ref[...] + a0_ref[...] + a1_ref[...]
    ms = jnp.mean(h * h, axis=-1, keepdims=True)
    o_ref[...] = h * lax.rsqrt(ms + EPS) * g_ref[...] + bt_ref[...]


def _norm(x, a0, a1, g2, beta2):
    return pl.pallas_call(
        _norm_body,
        grid=(10,),
        in_specs=[
            pl.BlockSpec((N // 10, D), lambda i: (i, 0)),
            pl.BlockSpec((N // 10, D), lambda i: (i, 0)),
            pl.BlockSpec((N // 10, D), lambda i: (i, 0)),
            pl.BlockSpec((1, D), lambda i: (0, 0)),
            pl.BlockSpec((1, D), lambda i: (0, 0)),
        ],
        out_specs=pl.BlockSpec((N // 10, D), lambda i: (i, 0)),
        out_shape=jax.ShapeDtypeStruct((N, D), jnp.float32),
    )(x, a0, a1, g2, beta2)


NB = 2
IB = 16
NIB = CHUNKS_PER_TILE // IB
INNER = IB // NB


@functools.partial(
    pl.kernel,
    out_type=jax.ShapeDtypeStruct((NC, N_SP, D), jnp.float32),
    mesh=plsc.VectorSubcoreMesh(core_axis_name="c", subcore_axis_name="s"),
    scratch_types=[
        pltpu.VMEM((IB, CHUNK), jnp.int32),
        pltpu.VMEM((IB, CHUNK), jnp.int32),
        pltpu.VMEM((NB, CHUNK, D), jnp.float32),
        pltpu.VMEM((ZROWS, D), jnp.float32),
        pltpu.VMEM_SHARED((N_SP, D), jnp.float32),
        pltpu.SemaphoreType.DMA((NB,)),
    ],
)
def _sc_agg(m_hbm, src_hbm, dst_hbm, out_hbm, idx_s, idx_d, rows, zbuf, shared, sem):
    c = lax.axis_index("c")
    s = lax.axis_index("s")
    wid = c * NS + s

    @pl.loop(0, ZROWS)
    def _zrow(r):
        @pl.loop(0, D, step=16)
        def _zcol(col):
            zbuf[r, pl.ds(col, 16)] = jnp.zeros((16,), jnp.float32)

    zbase = s * ROWS_PER_TILE_SP

    @pl.loop(0, (ROWS_PER_TILE_SP // ZROWS) * ZROWS, step=ZROWS)
    def _zspm(r0):
        pltpu.sync_copy(zbuf, shared.at[pl.ds(zbase + r0, ZROWS)])

    _rem = ROWS_PER_TILE_SP % ZROWS
    pltpu.sync_copy(
        zbuf.at[pl.ds(0, _rem)],
        shared.at[pl.ds(zbase + (ROWS_PER_TILE_SP // ZROWS) * ZROWS, _rem)],
    )

    plsc.subcore_barrier()

    qb = wid * CHUNKS_PER_TILE

    @pl.loop(0, NIB)
    def _blk(blk):
        pltpu.sync_copy(src_hbm.at[pl.ds(qb + blk * IB, IB)], idx_s)
        pltpu.sync_copy(dst_hbm.at[pl.ds(qb + blk * IB, IB)], idx_d)

        for b in range(NB):
            pltpu.async_copy(m_hbm.at[idx_s.at[b]], rows.at[b], sem.at[b])

        @pl.loop(0, INNER - 1)
        def _edge(i):
            for b in range(NB):
                pltpu.make_async_copy(
                    m_hbm.at[idx_s.at[b]], rows.at[b], sem.at[b]).wait()
                pltpu.sync_copy(rows.at[b], shared.at[idx_d.at[i * NB + b]],
                                add=True)
                pltpu.async_copy(
                    m_hbm.at[idx_s.at[(i + 1) * NB + b]], rows.at[b],
                    sem.at[b])

        for b in range(NB):
            pltpu.make_async_copy(
                m_hbm.at[idx_s.at[b]], rows.at[b], sem.at[b]).wait()
            pltpu.sync_copy(rows.at[b],
                            shared.at[idx_d.at[(INNER - 1) * NB + b]],
                            add=True)

    plsc.subcore_barrier()

    obase = s * ROWS_PER_TILE_SP
    pltpu.sync_copy(
        shared.at[pl.ds(obase, ROWS_PER_TILE_SP)],
        out_hbm.at[c, pl.ds(obase, ROWS_PER_TILE_SP)],
    )


def kernel(x, edge_index, W, b, g, beta):
    wt = W.T
    b2 = b.reshape(1, D)
    g2 = g.reshape(1, D)
    beta2 = beta.reshape(1, D)
    m = _linrelu(x, wt, b2)
    pad = E_PAD - E
    src = jnp.concatenate(
        [edge_index[0], jnp.zeros((pad,), jnp.int32)]).reshape(NUM_CHUNKS, CHUNK)
    dst = jnp.concatenate(
        [edge_index[1], jnp.full((pad,), N, jnp.int32)]).reshape(NUM_CHUNKS, CHUNK)
    parts = _sc_agg(m, src, dst)
    return _norm(x, parts[0, :N], parts[1, :N], g2, beta2)


# speedup vs baseline: 10.0121x; 2.9209x over previous
import functools

import jax
import jax.numpy as jnp
from jax import lax
from jax.experimental import pallas as pl
from jax.experimental.pallas import tpu as pltpu
from jax.experimental.pallas import tpu_sc as plsc

N = 10000
D = 128
E = 320000
EPS = 1e-5

CHUNK = 128
NC, NS = 2, 16
NW = NC * NS
NUM_CHUNKS = 2560
E_PAD = NUM_CHUNKS * CHUNK
CHUNKS_PER_TILE = NUM_CHUNKS // NW
ZROWS = 64
ROWS_PER_TILE_SP = 632
N_SP = NS * ROWS_PER_TILE_SP


def _linrelu_body(x_ref, wt_ref, b_ref, o_ref):
    acc = jnp.dot(x_ref[...], wt_ref[...], preferred_element_type=jnp.float32)
    o_ref[...] = jnp.maximum(acc + b_ref[...], 0.0)


def _linrelu(x, wt, b2):
    return pl.pallas_call(
        _linrelu_body,
        grid=(10,),
        in_specs=[
            pl.BlockSpec((N // 10, D), lambda i: (i, 0)),
            pl.BlockSpec((D, D), lambda i: (0, 0)),
            pl.BlockSpec((1, D), lambda i: (0, 0)),
        ],
        out_specs=pl.BlockSpec((N // 10, D), lambda i: (i, 0)),
        out_shape=jax.ShapeDtypeStruct((N, D), jnp.float32),
    )(x, wt, b2)


def _norm_body(x_ref, a0_ref, a1_ref, g_ref, bt_ref, o_ref):
    h = x_ref[...] + a0_ref[...] + a1_ref[...]
    ms = jnp.mean(h * h, axis=-1, keepdims=True)
    o_ref[...] = h * lax.rsqrt(ms + EPS) * g_ref[...] + bt_ref[...]


def _norm(x, a0, a1, g2, beta2):
    return pl.pallas_call(
        _norm_body,
        grid=(10,),
        in_specs=[
            pl.BlockSpec((N // 10, D), lambda i: (i, 0)),
            pl.BlockSpec((N // 10, D), lambda i: (i, 0)),
            pl.BlockSpec((N // 10, D), lambda i: (i, 0)),
            pl.BlockSpec((1, D), lambda i: (0, 0)),
            pl.BlockSpec((1, D), lambda i: (0, 0)),
        ],
        out_specs=pl.BlockSpec((N // 10, D), lambda i: (i, 0)),
        out_shape=jax.ShapeDtypeStruct((N, D), jnp.float32),
    )(x, a0, a1, g2, beta2)


NB = 2
IB = 16
NIB = CHUNKS_PER_TILE // IB
INNER = IB // NB


@functools.partial(
    pl.kernel,
    out_type=jax.ShapeDtypeStruct((NC, N_SP, D), jnp.float32),
    mesh=plsc.VectorSubcoreMesh(core_axis_name="c", subcore_axis_name="s"),
    scratch_types=[
        pltpu.VMEM((IB, CHUNK), jnp.int32),
        pltpu.VMEM((IB, CHUNK), jnp.int32),
        pltpu.VMEM((NB, CHUNK, D), jnp.float32),
        pltpu.VMEM((ZROWS, D), jnp.float32),
        pltpu.VMEM_SHARED((N_SP, D), jnp.float32),
        pltpu.SemaphoreType.DMA((NB,)),
    ],
)
def _sc_agg(m_hbm, src_hbm, dst_hbm, out_hbm, idx_s, idx_d, rows, zbuf, shared, sem):
    c = lax.axis_index("c")
    s = lax.axis_index("s")
    wid = c * NS + s

    @pl.loop(0, ZROWS)
    def _zrow(r):
        @pl.loop(0, D, step=16)
        def _zcol(col):
            zbuf[r, pl.ds(col, 16)] = jnp.zeros((16,), jnp.float32)

    zbase = s * ROWS_PER_TILE_SP

    @pl.loop(0, (ROWS_PER_TILE_SP // ZROWS) * ZROWS, step=ZROWS)
    def _zspm(r0):
        pltpu.sync_copy(zbuf, shared.at[pl.ds(zbase + r0, ZROWS)])

    _rem = ROWS_PER_TILE_SP % ZROWS
    pltpu.sync_copy(
        zbuf.at[pl.ds(0, _rem)],
        shared.at[pl.ds(zbase + (ROWS_PER_TILE_SP // ZROWS) * ZROWS, _rem)],
    )

    plsc.subcore_barrier()

    qb = wid * CHUNKS_PER_TILE

    @pl.loop(0, NIB)
    def _blk(blk):
        pltpu.sync_copy(src_hbm.at[pl.ds(qb + blk * IB, IB)], idx_s)
        pltpu.sync_copy(dst_hbm.at[pl.ds(qb + blk * IB, IB)], idx_d)

        for b in range(NB):
            pltpu.async_copy(m_hbm.at[idx_s.at[b]], rows.at[b], sem.at[b])

        @pl.loop(0, INNER - 1)
        def _edge(i):
            for b in range(NB):
                pltpu.make_async_copy(
                    m_hbm.at[idx_s.at[b]], rows.at[b], sem.at[b]).wait()
                pltpu.sync_copy(rows.at[b], shared.at[idx_d.at[i * NB + b]],
                                add=True)
                pltpu.async_copy(
                    m_hbm.at[idx_s.at[(i + 1) * NB + b]], rows.at[b],
                    sem.at[b])

        for b in range(NB):
            pltpu.make_async_copy(
                m_hbm.at[idx_s.at[b]], rows.at[b], sem.at[b]).wait()
            pltpu.sync_copy(rows.at[b],
                            shared.at[idx_d.at[(INNER - 1) * NB + b]],
                            add=True)

    plsc.subcore_barrier()

    obase = s * ROWS_PER_TILE_SP
    pltpu.sync_copy(
        shared.at[pl.ds(obase, ROWS_PER_TILE_SP)],
        out_hbm.at[c, pl.ds(obase, ROWS_PER_TILE_SP)],
    )


def kernel(x, edge_index, W, b, g, beta):
    wt = W.T
    b2 = b.reshape(1, D)
    g2 = g.reshape(1, D)
    beta2 = beta.reshape(1, D)
    m = _linrelu(x, wt, b2)
    pad = E_PAD - E
    pad_iota = jnp.arange(pad, dtype=jnp.int32)
    src = jnp.concatenate(
        [edge_index[0], pad_iota % N]).reshape(NUM_CHUNKS, CHUNK)
    dst = jnp.concatenate(
        [edge_index[1], N + pad_iota % (N_SP - N)]).reshape(NUM_CHUNKS, CHUNK)
    parts = _sc_agg(m, src, dst)
    return _norm(x, parts[0, :N], parts[1, :N], g2, beta2)


# speedup vs baseline: 10.9856x; 1.0972x over previous
import functools

import jax
import jax.numpy as jnp
from jax import lax
from jax.experimental import pallas as pl
from jax.experimental.pallas import tpu as pltpu
from jax.experimental.pallas import tpu_sc as plsc

N = 10000
D = 128
E = 320000
EPS = 1e-5

CHUNK = 128
NC, NS = 2, 16
NW = NC * NS
NUM_CHUNKS = 2560
E_PAD = NUM_CHUNKS * CHUNK
CHUNKS_PER_TILE = NUM_CHUNKS // NW
ZROWS = 64
ROWS_PER_TILE_SP = 632
N_SP = NS * ROWS_PER_TILE_SP


def _linrelu_body(x_ref, wt_ref, b_ref, o_ref):
    acc = jnp.dot(x_ref[...], wt_ref[...], preferred_element_type=jnp.float32)
    o_ref[...] = jnp.maximum(acc + b_ref[...], 0.0)


def _linrelu(x, wt, b2):
    return pl.pallas_call(
        _linrelu_body,
        grid=(10,),
        in_specs=[
            pl.BlockSpec((N // 10, D), lambda i: (i, 0)),
            pl.BlockSpec((D, D), lambda i: (0, 0)),
            pl.BlockSpec((1, D), lambda i: (0, 0)),
        ],
        out_specs=pl.BlockSpec((N // 10, D), lambda i: (i, 0)),
        out_shape=jax.ShapeDtypeStruct((N, D), jnp.float32),
    )(x, wt, b2)


def _norm_body(x_ref, parts_ref, g_ref, bt_ref, o_ref):
    h = x_ref[...] + parts_ref[0] + parts_ref[1]
    ms = jnp.mean(h * h, axis=-1, keepdims=True)
    o_ref[...] = h * lax.rsqrt(ms + EPS) * g_ref[...] + bt_ref[...]


def _norm(x, parts, g2, beta2):
    return pl.pallas_call(
        _norm_body,
        grid=(10,),
        in_specs=[
            pl.BlockSpec((N // 10, D), lambda i: (i, 0)),
            pl.BlockSpec((NC, N // 10, D), lambda i: (0, i, 0)),
            pl.BlockSpec((1, D), lambda i: (0, 0)),
            pl.BlockSpec((1, D), lambda i: (0, 0)),
        ],
        out_specs=pl.BlockSpec((N // 10, D), lambda i: (i, 0)),
        out_shape=jax.ShapeDtypeStruct((N, D), jnp.float32),
    )(x, parts, g2, beta2)


NB = 2
IB = 40
NIB = CHUNKS_PER_TILE // IB
INNER = IB // NB


@functools.partial(
    pl.kernel,
    out_type=jax.ShapeDtypeStruct((NC, N_SP, D), jnp.float32),
    mesh=plsc.VectorSubcoreMesh(core_axis_name="c", subcore_axis_name="s"),
    scratch_types=[
        pltpu.VMEM((IB, CHUNK), jnp.int32),
        pltpu.VMEM((IB, CHUNK), jnp.int32),
        pltpu.VMEM((NB, CHUNK, D), jnp.float32),
        pltpu.VMEM_SHARED((N_SP, D), jnp.float32),
        pltpu.SemaphoreType.DMA((NB,)),
    ],
)
def _sc_agg(m_hbm, src_hbm, dst_hbm, out_hbm, idx_s, idx_d, rows, shared, sem):
    c = lax.axis_index("c")
    s = lax.axis_index("s")
    wid = c * NS + s

    @pl.loop(0, CHUNK)
    def _zrow(r):
        @pl.loop(0, D, step=16)
        def _zcol(col):
            rows[0, r, pl.ds(col, 16)] = jnp.zeros((16,), jnp.float32)

    zbase = s * ROWS_PER_TILE_SP

    @pl.loop(0, (ROWS_PER_TILE_SP // CHUNK) * CHUNK, step=CHUNK)
    def _zspm(r0):
        pltpu.sync_copy(rows.at[0], shared.at[pl.ds(zbase + r0, CHUNK)])

    _rem = ROWS_PER_TILE_SP % CHUNK
    pltpu.sync_copy(
        rows.at[0].at[pl.ds(0, _rem)],
        shared.at[pl.ds(zbase + (ROWS_PER_TILE_SP // CHUNK) * CHUNK, _rem)],
    )

    plsc.subcore_barrier()

    qb = wid * CHUNKS_PER_TILE

    @pl.loop(0, NIB)
    def _blk(blk):
        pltpu.sync_copy(src_hbm.at[pl.ds(qb + blk * IB, IB)], idx_s)
        pltpu.sync_copy(dst_hbm.at[pl.ds(qb + blk * IB, IB)], idx_d)

        for b in range(NB):
            pltpu.async_copy(m_hbm.at[idx_s.at[b]], rows.at[b], sem.at[b])

        @pl.loop(0, INNER - 1)
        def _edge(i):
            for b in range(NB):
                pltpu.make_async_copy(
                    m_hbm.at[idx_s.at[b]], rows.at[b], sem.at[b]).wait()
                pltpu.sync_copy(rows.at[b], shared.at[idx_d.at[i * NB + b]],
                                add=True)
                pltpu.async_copy(
                    m_hbm.at[idx_s.at[(i + 1) * NB + b]], rows.at[b],
                    sem.at[b])

        for b in range(NB):
            pltpu.make_async_copy(
                m_hbm.at[idx_s.at[b]], rows.at[b], sem.at[b]).wait()
            pltpu.sync_copy(rows.at[b],
                            shared.at[idx_d.at[(INNER - 1) * NB + b]],
                            add=True)

    plsc.subcore_barrier()

    obase = s * ROWS_PER_TILE_SP
    pltpu.sync_copy(
        shared.at[pl.ds(obase, ROWS_PER_TILE_SP)],
        out_hbm.at[c, pl.ds(obase, ROWS_PER_TILE_SP)],
    )


def kernel(x, edge_index, W, b, g, beta):
    wt = W.T
    b2 = b.reshape(1, D)
    g2 = g.reshape(1, D)
    beta2 = beta.reshape(1, D)
    m = _linrelu(x, wt, b2)
    pad = E_PAD - E
    pad_iota = jnp.arange(pad, dtype=jnp.int32)
    src = jnp.concatenate(
        [edge_index[0], pad_iota % N]).reshape(NUM_CHUNKS, CHUNK)
    dst = jnp.concatenate(
        [edge_index[1], N + pad_iota % (N_SP - N)]).reshape(NUM_CHUNKS, CHUNK)
    parts = _sc_agg(m, src, dst)
    return _norm(x, parts, g2, beta2)
